# Initial kernel scaffold; baseline (speedup 1.0000x reference)
import jax
import jax.numpy as jnp
from jax.experimental import pallas as pl


def kernel(pred_tensor, target_tensor):
    raise NotImplementedError("write your pallas kernel here")


# speedup vs baseline: 1.6922x; 1.6922x over previous
import jax
import jax.numpy as jnp
from jax.experimental import pallas as pl

B_BOX = 2
C_CLS = 20
LAMBDA_COORD = 5.0
LAMBDA_NOOBJ = 0.5
N_ELEM = B_BOX * 5 + C_CLS
BATCH = 4096
S = 7
N_CELLS = BATCH * S * S

GRID = 32
ROWS = N_CELLS // GRID


def _iou(bp, bt):
    lt = jnp.maximum(bp[:, 0:2], bt[:, 0:2])
    rb = jnp.minimum(bp[:, 2:4], bt[:, 2:4])
    wh = jnp.clip(rb - lt, 0.0, None)
    inter = wh[:, 0:1] * wh[:, 1:2]
    a1 = (bp[:, 2:3] - bp[:, 0:1]) * (bp[:, 3:4] - bp[:, 1:2])
    a2 = (bt[:, 2:3] - bt[:, 0:1]) * (bt[:, 3:4] - bt[:, 1:2])
    return inter / (a1 + a2 - inter)


def _block_body(p_ref, t_ref, o_ref):
    p = p_ref[...]
    t = t_ref[...]
    d2 = (p - t) ** 2

    t5 = t[:, 5:6]
    coord = (t5 > 0).astype(jnp.float32)
    noobj = (t5 == 0).astype(jnp.float32)

    class_sum = jnp.sum(d2[:, 10:30], axis=1, keepdims=True)
    s0 = jnp.sum(d2[:, 0:4], axis=1, keepdims=True)
    s1 = jnp.sum(d2[:, 5:9], axis=1, keepdims=True)
    c0 = d2[:, 4:5]
    c1 = d2[:, 9:10]

    bp0, bp1 = p[:, 0:4], p[:, 5:9]
    bt0, bt1 = t[:, 0:4], t[:, 5:9]
    m0 = _iou(bp1, bt0) > _iou(bp0, bt0)
    m1 = _iou(bp1, bt1) > _iou(bp0, bt1)
    resp0 = ((~m0) | (~m1)).astype(jnp.float32) * coord
    resp1 = (m0 | m1).astype(jnp.float32) * coord

    per_cell = (
        LAMBDA_COORD * (resp0 * s0 + resp1 * s1)
        + resp0 * c0 + resp1 * c1
        + LAMBDA_NOOBJ * noobj * (c0 + c1)
        + coord * class_sum
    )
    partial = jnp.sum(per_cell).reshape(1, 1)

    @pl.when(pl.program_id(0) == 0)
    def _():
        o_ref[...] = jnp.zeros((1, 1), jnp.float32)

    o_ref[...] += partial


def kernel(pred_tensor, target_tensor):
    p = pred_tensor.reshape(N_CELLS, N_ELEM)
    t = target_tensor.reshape(N_CELLS, N_ELEM)
    out = pl.pallas_call(
        _block_body,
        grid=(GRID,),
        in_specs=[
            pl.BlockSpec((ROWS, N_ELEM), lambda i: (i, 0)),
            pl.BlockSpec((ROWS, N_ELEM), lambda i: (i, 0)),
        ],
        out_specs=pl.BlockSpec((1, 1), lambda i: (0, 0)),
        out_shape=jax.ShapeDtypeStruct((1, 1), jnp.float32),
    )(p, t)
    return out[0, 0]


# speedup vs baseline: 3.4082x; 2.0140x over previous
import jax
import jax.numpy as jnp
from jax.experimental import pallas as pl

B_BOX = 2
C_CLS = 20
LAMBDA_COORD = 5.0
LAMBDA_NOOBJ = 0.5
N_ELEM = B_BOX * 5 + C_CLS
BATCH = 4096
S = 7
N_CELLS = BATCH * S * S

PACK = 4
W = PACK * N_ELEM
N4 = N_CELLS // PACK
GRID = 28
ROWS = N4 // GRID


def _rl(x, k):
    return jnp.roll(x, -k, axis=1)


def _block_body(p_ref, t_ref, o_ref):
    x = p_ref[...]
    y = t_ref[...]

    d = x - y
    d2 = d * d
    s1 = d2 + _rl(d2, 1)
    s2 = s1 + _rl(s1, 2)
    s4 = s2 + _rl(s2, 4)
    s8 = s4 + _rl(s4, 8)

    s_box0 = s2
    s_box1 = _rl(s2, 5)
    s_class = _rl(s8, 10) + _rl(s2, 26)
    c0 = _rl(d2, 4)
    c1 = _rl(d2, 9)

    t5 = _rl(y, 5)
    coord = t5 > 0.0
    noobj = t5 == 0.0
    coordf = jnp.where(coord, 1.0, 0.0).astype(jnp.float32)
    noobjf = jnp.where(noobj, 1.0, 0.0).astype(jnp.float32)

    ex = _rl(x, 2) - x
    area_p = ex * _rl(ex, 1)
    ey = _rl(y, 2) - y
    area_t = ey * _rl(ey, 1)
    x5 = _rl(x, 5)
    ap0, ap1 = area_p, _rl(area_p, 5)
    at0, at1 = area_t, _rl(area_t, 5)

    def iou(a, b, aa, ab):
        mx = jnp.maximum(a, b)
        mn = jnp.minimum(a, b)
        wh = jnp.maximum(_rl(mn, 2) - mx, 0.0)
        inter = wh * _rl(wh, 1)
        return inter / (aa + ab - inter)

    iou00 = iou(x, y, ap0, at0)
    iou10 = iou(x5, y, ap1, at0)
    iou01 = iou(x, t5, ap0, at1)
    iou11 = iou(x5, t5, ap1, at1)
    m0 = iou10 > iou00
    m1 = iou11 > iou01
    r0 = jnp.where(~m0 | ~m1, coordf, 0.0)
    r1 = jnp.where(m0 | m1, coordf, 0.0)

    per_cell = (
        LAMBDA_COORD * (r0 * s_box0 + r1 * s_box1)
        + r0 * c0 + r1 * c1
        + LAMBDA_NOOBJ * noobjf * (c0 + c1)
        + coordf * s_class
    )
    lane = jax.lax.broadcasted_iota(jnp.int32, per_cell.shape, 1)
    masked = jnp.where(lane % N_ELEM == 0, per_cell, 0.0)
    partial = jnp.sum(masked).reshape(1, 1)

    @pl.when(pl.program_id(0) == 0)
    def _():
        o_ref[...] = jnp.zeros((1, 1), jnp.float32)

    o_ref[...] += partial


def kernel(pred_tensor, target_tensor):
    p = pred_tensor.reshape(N4, W)
    t = target_tensor.reshape(N4, W)
    out = pl.pallas_call(
        _block_body,
        grid=(GRID,),
        in_specs=[
            pl.BlockSpec((ROWS, W), lambda i: (i, 0)),
            pl.BlockSpec((ROWS, W), lambda i: (i, 0)),
        ],
        out_specs=pl.BlockSpec((1, 1), lambda i: (0, 0)),
        out_shape=jax.ShapeDtypeStruct((1, 1), jnp.float32),
    )(p, t)
    return out[0, 0]
